# Initial kernel scaffold; baseline (speedup 1.0000x reference)
import jax
import jax.numpy as jnp
from jax.experimental import pallas as pl


def kernel(x, edge_index, edge_attr, h_prev, W0, b0, W1, b1, W_mix, b_mix, W_ih, W_hh, b_ih, b_hh, W_ro, b_ro):
    raise NotImplementedError("write your pallas kernel here")


# speedup vs baseline: 15.1163x; 15.1163x over previous
import functools

import jax
import jax.numpy as jnp
from jax import lax
from jax.experimental import pallas as pl
from jax.experimental.pallas import tpu as pltpu
from jax.experimental.pallas import tpu_sc as plsc

N = 10000
E = 320000
TWO_N = 2 * N
HID = 128
QCOL = 32
NQ = HID // QCOL
OUT_DIM = 64

NUM_CORES = 2
NUM_TILES = 16
CHUNK = 400
VECS = CHUNK // 16
EDGES_PER_TILE = E // NUM_TILES
CHUNKS_PER_TILE = EDGES_PER_TILE // CHUNK
H_SLICE = 1248
H_TAIL_OFF = H_SLICE * NUM_TILES
H_TAIL = TWO_N - H_TAIL_OFF
B_ROWS = 416


def _sc_pass_body(gather_by_edge, with_hist, *refs):
    if with_hist:
        (ni_hbm, ee_hbm, at_hbm, tab0, tab1, tab2, tab3,
         out0, out1, out2, out3, histA, histB,
         bn, be, bt, bgi, bsi, onesb, hbuf, rows_v, bounce,
         acc_sh, hist_sh, sem) = refs
    else:
        (ni_hbm, ee_hbm, at_hbm, tab0, tab1, tab2, tab3,
         out0, out1, out2, out3,
         bn, be, bt, bgi, bsi, rows_v, bounce, acc_sh, sem) = refs

    cid = lax.axis_index("c")
    wid = lax.axis_index("s")
    r0 = wid * H_SLICE
    ebase = wid * EDGES_PER_TILE

    zeros16 = jnp.zeros((16,), jnp.float32)

    def zb_body(i, c):
        r = i // 2
        j = (i % 2) * 16
        bounce[r, pl.ds(j, 16)] = zeros16
        return c

    lax.fori_loop(0, B_ROWS * (QCOL // 16), zb_body, 0)

    if with_hist:
        def zh_body(i, c):
            hbuf[pl.ds(i * 16, 16)] = zeros16
            return c

        lax.fori_loop(0, H_SLICE // 16, zh_body, 0)
        pltpu.sync_copy(hbuf, hist_sh.at[pl.ds(r0, H_SLICE)])

        @pl.when(wid == 0)
        def _():
            pltpu.sync_copy(hbuf.at[pl.ds(0, H_TAIL)],
                            hist_sh.at[pl.ds(H_TAIL_OFF, H_TAIL)])

        ones16 = jnp.ones((16,), jnp.float32)

        def ob_body(i, c):
            onesb[pl.ds(i * 16, 16)] = ones16
            return c

        lax.fori_loop(0, VECS, ob_body, 0)

    def round_q(q, tab, out_ref, do_hist):
        def zacc_body(i, c):
            pltpu.sync_copy(bounce, acc_sh.at[pl.ds(r0 + i * B_ROWS, B_ROWS)])
            return c

        lax.fori_loop(0, H_SLICE // B_ROWS, zacc_body, 0)

        @pl.when(wid == 0)
        def _():
            pltpu.sync_copy(bounce.at[pl.ds(0, H_TAIL)],
                            acc_sh.at[pl.ds(H_TAIL_OFF, H_TAIL)])

        plsc.subcore_barrier()

        def chunk_body(k, carry):
            base = ebase + k * CHUNK
            pltpu.sync_copy(ni_hbm.at[pl.ds(base, CHUNK)], bn)
            pltpu.sync_copy(ee_hbm.at[pl.ds(base, CHUNK)], be)
            pltpu.sync_copy(at_hbm.at[pl.ds(base, CHUNK)], bt)

            def vec_body(i, c2):
                s = i * 16
                off = bt[pl.ds(s, 16)] * N
                bgi[pl.ds(s, 16)] = bn[pl.ds(s, 16)] + off
                bsi[pl.ds(s, 16)] = be[pl.ds(s, 16)] + off
                return c2

            lax.fori_loop(0, VECS, vec_body, 0)

            g_idx = bsi if gather_by_edge else bgi
            s_idx = bgi if gather_by_edge else bsi

            pltpu.async_copy(tab.at[g_idx], rows_v, sem).wait()
            pltpu.sync_copy(rows_v, acc_sh.at[s_idx], add=True)

            if do_hist:
                @pl.when(cid == 0)
                def _():
                    pltpu.sync_copy(onesb, hist_sh.at[bgi], add=True)

                @pl.when(cid == 1)
                def _():
                    pltpu.sync_copy(onesb, hist_sh.at[bsi], add=True)
            return carry

        lax.fori_loop(0, CHUNKS_PER_TILE, chunk_body, 0)
        plsc.subcore_barrier()

        def wb_body(i, c):
            off = r0 + i * B_ROWS
            pltpu.sync_copy(acc_sh.at[pl.ds(off, B_ROWS)], bounce)
            pltpu.sync_copy(bounce, out_ref.at[pl.ds(off, B_ROWS)])
            return c

        lax.fori_loop(0, H_SLICE // B_ROWS, wb_body, 0)

        @pl.when(wid == 0)
        def _():
            pltpu.sync_copy(acc_sh.at[pl.ds(H_TAIL_OFF, H_TAIL)],
                            bounce.at[pl.ds(0, H_TAIL)])
            pltpu.sync_copy(bounce.at[pl.ds(0, H_TAIL)],
                            out_ref.at[pl.ds(H_TAIL_OFF, H_TAIL)])

        lax.fori_loop(0, B_ROWS * (QCOL // 16), zb_body, 0)

    @pl.when(cid == 0)
    def _():
        round_q(0, tab0, out0, with_hist)
        round_q(1, tab1, out1, False)

    @pl.when(cid == 1)
    def _():
        round_q(0, tab2, out2, with_hist)
        round_q(1, tab3, out3, False)

    if with_hist:
        plsc.subcore_barrier()

        def _hist_out(h_ref):
            pltpu.sync_copy(hist_sh.at[pl.ds(r0, H_SLICE)], hbuf)
            pltpu.sync_copy(hbuf, h_ref.at[pl.ds(r0, H_SLICE)])

            @pl.when(wid == 0)
            def _():
                pltpu.sync_copy(hist_sh.at[pl.ds(H_TAIL_OFF, H_TAIL)],
                                hbuf.at[pl.ds(0, H_TAIL)])
                pltpu.sync_copy(hbuf.at[pl.ds(0, H_TAIL)],
                                h_ref.at[pl.ds(H_TAIL_OFF, H_TAIL)])

        @pl.when(cid == 0)
        def _():
            _hist_out(histA)

        @pl.when(cid == 1)
        def _():
            _hist_out(histB)


def _make_sc_pass(gather_by_edge, with_hist):
    mesh = plsc.VectorSubcoreMesh(core_axis_name="c", subcore_axis_name="s",
                                  num_cores=NUM_CORES, num_subcores=NUM_TILES)
    f32, i32 = jnp.float32, jnp.int32
    out_type = [jax.ShapeDtypeStruct((TWO_N, QCOL), f32) for _ in range(NQ)]
    scratch = [pltpu.VMEM((CHUNK,), i32),
               pltpu.VMEM((CHUNK,), i32),
               pltpu.VMEM((CHUNK,), i32),
               pltpu.VMEM((CHUNK,), i32),
               pltpu.VMEM((CHUNK,), i32),
               ]
    if with_hist:
        out_type += [jax.ShapeDtypeStruct((TWO_N,), f32),
                     jax.ShapeDtypeStruct((TWO_N,), f32)]
        scratch += [pltpu.VMEM((CHUNK,), f32),
                    pltpu.VMEM((H_SLICE,), f32)]
    scratch += [pltpu.VMEM((CHUNK, QCOL), f32)]
    scratch += [pltpu.VMEM((B_ROWS, QCOL), f32)]
    scratch += [pltpu.VMEM_SHARED((TWO_N, QCOL), f32)]
    if with_hist:
        scratch += [pltpu.VMEM_SHARED((TWO_N,), f32)]
    scratch += [pltpu.SemaphoreType.DMA]

    body = functools.partial(_sc_pass_body, gather_by_edge, with_hist)
    return pl.kernel(body, out_type=out_type, mesh=mesh,
                     scratch_types=scratch,
                     compiler_params=pltpu.CompilerParams(
                         use_tc_tiling_on_sc=False))


_sc_pass1 = _make_sc_pass(gather_by_edge=False, with_hist=True)
_sc_pass2 = _make_sc_pass(gather_by_edge=True, with_hist=False)


def _xl_body(x_ref, W_ref, o0_ref, o1_ref, o2_ref, o3_ref):
    res = lax.dot_general(x_ref[...], W_ref[0],
                          (((1,), (1,)), ((), ())),
                          preferred_element_type=jnp.float32)
    o0_ref[0] = res[:, 0 * QCOL:1 * QCOL]
    o1_ref[0] = res[:, 1 * QCOL:2 * QCOL]
    o2_ref[0] = res[:, 2 * QCOL:3 * QCOL]
    o3_ref[0] = res[:, 3 * QCOL:4 * QCOL]


def _xl_quarters(x, W01):
    nb = 25
    blk = N // nb
    outs = pl.pallas_call(
        _xl_body,
        grid=(nb, 2),
        in_specs=[pl.BlockSpec((blk, HID), lambda i, t: (i, 0)),
                  pl.BlockSpec((1, HID, HID), lambda i, t: (t, 0, 0))],
        out_specs=[pl.BlockSpec((1, blk, QCOL), lambda i, t: (t, i, 0))
                   for _ in range(NQ)],
        out_shape=[jax.ShapeDtypeStruct((2, N, QCOL), jnp.float32)
                   for _ in range(NQ)],
    )(x, W01)
    return [o.reshape(TWO_N, QCOL) for o in outs]


def _scale_body(e0, e1, e2, e3, bd_ref, dd_ref,
                o0, o1, o2, o3, dinv_ref):
    bd = bd_ref[0, 0, :]
    binv = jnp.where(bd > 0, 1.0 / bd, 0.0).reshape(-1, 1)
    o0[...] = e0[...] * binv
    o1[...] = e1[...] * binv
    o2[...] = e2[...] * binv
    o3[...] = e3[...] * binv
    dd = dd_ref[0, 0, :]
    dinv_ref[0, 0, :] = jnp.where(dd > 0, 1.0 / dd, 0.0)


def _scale_ef(efq, Bd, Dd):
    nb = 50
    blk = TWO_N // nb
    res = pl.pallas_call(
        _scale_body,
        grid=(nb,),
        in_specs=[pl.BlockSpec((blk, QCOL), lambda i: (i, 0))
                  for _ in range(NQ)]
        + [pl.BlockSpec((1, 1, blk), lambda i: (i, 0, 0)),
           pl.BlockSpec((1, 1, blk), lambda i: (i, 0, 0))],
        out_specs=[pl.BlockSpec((blk, QCOL), lambda i: (i, 0))
                   for _ in range(NQ)]
        + [pl.BlockSpec((1, 1, blk), lambda i: (i, 0, 0))],
        out_shape=[jax.ShapeDtypeStruct((TWO_N, QCOL), jnp.float32)
                   for _ in range(NQ)]
        + [jax.ShapeDtypeStruct((nb, 1, blk), jnp.float32)],
    )(*efq, Bd.reshape(nb, 1, blk), Dd.reshape(nb, 1, blk))
    return res[:NQ], res[NQ]


def _final_body(q00, q01, q02, q03, q10, q11, q12, q13, d0, d1, hp_ref,
                Wm0T, Wm1T, bh, WihT, bih, WhhT, bhh, WroT, bro,
                hn_ref, pr_ref):
    dinv0 = d0[0, 0, :].reshape(-1, 1)
    dinv1 = d1[0, 0, :].reshape(-1, 1)
    x0 = jnp.concatenate([q00[...], q01[...], q02[...], q03[...]],
                         axis=1) * dinv0
    x1 = jnp.concatenate([q10[...], q11[...], q12[...], q13[...]],
                         axis=1) * dinv1
    pre = (jnp.dot(x0, Wm0T[...], preferred_element_type=jnp.float32)
           + jnp.dot(x1, Wm1T[...], preferred_element_type=jnp.float32)
           + bh[...])
    h = jnp.maximum(pre, 0.0)
    gi = jnp.dot(h, WihT[...], preferred_element_type=jnp.float32) + bih[...]
    hp = hp_ref[...]
    gh = jnp.dot(hp, WhhT[...], preferred_element_type=jnp.float32) + bhh[...]
    r = jax.nn.sigmoid(gi[:, :HID] + gh[:, :HID])
    z = jax.nn.sigmoid(gi[:, HID:2 * HID] + gh[:, HID:2 * HID])
    n = jnp.tanh(gi[:, 2 * HID:] + r * gh[:, 2 * HID:])
    hn = (1.0 - z) * n + z * hp
    hn_ref[...] = hn
    pr_ref[...] = jnp.dot(hn, WroT[...], preferred_element_type=jnp.float32) + bro[...]


def _final(outq, Dinv3, h_prev, Wm0T, Wm1T, bias_h,
           WihT, b_ih, WhhT, b_hh, WroT, b_ro):
    nb = 25
    blk = N // nb
    full = lambda shape: pl.BlockSpec(shape, lambda i: (0,) * len(shape))
    return pl.pallas_call(
        _final_body,
        grid=(nb,),
        in_specs=[pl.BlockSpec((blk, QCOL), lambda i: (i, 0))
                  for _ in range(NQ)]
        + [pl.BlockSpec((blk, QCOL), lambda i: (nb + i, 0))
           for _ in range(NQ)]
        + [pl.BlockSpec((1, 1, blk), lambda i: (i, 0, 0)),
           pl.BlockSpec((1, 1, blk), lambda i: (nb + i, 0, 0)),
           pl.BlockSpec((blk, HID), lambda i: (i, 0)),
           full((HID, HID)), full((HID, HID)), full((1, HID)),
           full((HID, 3 * HID)), full((1, 3 * HID)),
           full((HID, 3 * HID)), full((1, 3 * HID)),
           full((HID, OUT_DIM)), full((1, OUT_DIM))],
        out_specs=[pl.BlockSpec((blk, HID), lambda i: (i, 0)),
                   pl.BlockSpec((blk, OUT_DIM), lambda i: (i, 0))],
        out_shape=[jax.ShapeDtypeStruct((N, HID), jnp.float32),
                   jax.ShapeDtypeStruct((N, OUT_DIM), jnp.float32)],
    )(*outq, *outq, Dinv3, Dinv3, h_prev, Wm0T, Wm1T, bias_h,
      WihT, b_ih, WhhT, b_hh, WroT, b_ro)


def kernel(x, edge_index, edge_attr, h_prev, W0, b0, W1, b1, W_mix, b_mix,
           W_ih, W_hh, b_ih, b_hh, W_ro, b_ro):
    node_idx = edge_index[0]
    hedge_idx = edge_index[1]

    xlq = _xl_quarters(x, jnp.stack([W0, W1]))

    res1 = _sc_pass1(node_idx, hedge_idx, edge_attr, *xlq)
    efq, Dd, Bd = res1[:NQ], res1[NQ], res1[NQ + 1]

    efsq, Dinv3 = _scale_ef(efq, Bd, Dd)

    outq = _sc_pass2(node_idx, hedge_idx, edge_attr, *efsq)

    bias_h = (b0 @ W_mix[:, :HID].T + b1 @ W_mix[:, HID:].T + b_mix)
    h_next, pred = _final(
        outq, Dinv3, h_prev,
        W_mix[:, :HID].T, W_mix[:, HID:].T, bias_h.reshape(1, HID),
        W_ih.T, b_ih.reshape(1, 3 * HID), W_hh.T, b_hh.reshape(1, 3 * HID),
        W_ro.T, b_ro.reshape(1, OUT_DIM))
    return (h_next, pred[:, :3])


# speedup vs baseline: 27.7716x; 1.8372x over previous
import functools

import jax
import jax.numpy as jnp
from jax import lax
from jax.experimental import pallas as pl
from jax.experimental.pallas import tpu as pltpu
from jax.experimental.pallas import tpu_sc as plsc

N = 10000
E = 320000
TWO_N = 2 * N
HID = 128
QCOL = 32
NQ = HID // QCOL
OUT_DIM = 64

NUM_CORES = 2
NUM_TILES = 16
CHUNK = 400
VECS = CHUNK // 16
EDGES_PER_TILE = E // NUM_TILES
CHUNKS_PER_TILE = EDGES_PER_TILE // CHUNK
H_SLICE = 1248
H_TAIL_OFF = H_SLICE * NUM_TILES
H_TAIL = TWO_N - H_TAIL_OFF
B_ROWS = 416


def _sc_pass_body(gather_by_edge, with_hist, *refs):
    if with_hist:
        (ni_hbm, ee_hbm, at_hbm, tab0, tab1, tab2, tab3,
         out0, out1, out2, out3, histA, histB,
         bgi, bsi, onesb, hbuf, rows0, rows1, bounce,
         acc_sh, hist_sh, sem0, sem1) = refs
    else:
        (ni_hbm, ee_hbm, at_hbm, tab0, tab1, tab2, tab3,
         out0, out1, out2, out3,
         bgi, bsi, rows0, rows1, bounce, acc_sh, sem0, sem1) = refs

    cid = lax.axis_index("c")
    wid = lax.axis_index("s")
    r0 = wid * H_SLICE
    ebase = wid * EDGES_PER_TILE

    d0 = pltpu.async_copy(ni_hbm.at[pl.ds(ebase, EDGES_PER_TILE)], bgi, sem0)
    d1 = pltpu.async_copy(at_hbm.at[pl.ds(ebase, EDGES_PER_TILE)], bsi, sem1)

    zeros16 = jnp.zeros((16,), jnp.float32)

    def zb_body(i, c):
        r = i // 2
        j = (i % 2) * 16
        bounce[r, pl.ds(j, 16)] = zeros16
        return c

    lax.fori_loop(0, B_ROWS * (QCOL // 16), zb_body, 0)

    if with_hist:
        def zh_body(i, c):
            hbuf[pl.ds(i * 16, 16)] = zeros16
            return c

        lax.fori_loop(0, H_SLICE // 16, zh_body, 0)
        pltpu.sync_copy(hbuf, hist_sh.at[pl.ds(r0, H_SLICE)])

        @pl.when(wid == 0)
        def _():
            pltpu.sync_copy(hbuf.at[pl.ds(0, H_TAIL)],
                            hist_sh.at[pl.ds(H_TAIL_OFF, H_TAIL)])

        ones16 = jnp.ones((16,), jnp.float32)

        def ob_body(i, c):
            onesb[pl.ds(i * 16, 16)] = ones16
            return c

        lax.fori_loop(0, VECS, ob_body, 0)

    d0.wait()
    d1.wait()

    def ix1_body(i, c):
        s = i * 16
        bgi[pl.ds(s, 16)] = bgi[pl.ds(s, 16)] + bsi[pl.ds(s, 16)] * N
        return c

    lax.fori_loop(0, EDGES_PER_TILE // 16, ix1_body, 0)

    pltpu.async_copy(ee_hbm.at[pl.ds(ebase, EDGES_PER_TILE)], bsi, sem0).wait()

    def ix2_body(i, c):
        s = i * 16
        g = bgi[pl.ds(s, 16)]
        off = jnp.where(g >= N, N, 0).astype(jnp.int32)
        bsi[pl.ds(s, 16)] = bsi[pl.ds(s, 16)] + off
        return c

    lax.fori_loop(0, EDGES_PER_TILE // 16, ix2_body, 0)

    g_all = bsi if gather_by_edge else bgi
    s_all = bgi if gather_by_edge else bsi

    def round_q(q, tab, out_ref, do_hist):
        def zacc_body(i, c):
            pltpu.sync_copy(bounce, acc_sh.at[pl.ds(r0 + i * B_ROWS, B_ROWS)])
            return c

        lax.fori_loop(0, H_SLICE // B_ROWS, zacc_body, 0)

        @pl.when(wid == 0)
        def _():
            pltpu.sync_copy(bounce.at[pl.ds(0, H_TAIL)],
                            acc_sh.at[pl.ds(H_TAIL_OFF, H_TAIL)])

        plsc.subcore_barrier()

        def fire(c, rows, sem):
            pltpu.async_copy(tab.at[g_all.at[pl.ds(c * CHUNK, CHUNK)]],
                             rows, sem)

        def wait_g(c, rows, sem):
            pltpu.make_async_copy(tab.at[g_all.at[pl.ds(c * CHUNK, CHUNK)]],
                                  rows, sem).wait()

        def scatter(c, rows):
            pltpu.sync_copy(rows, acc_sh.at[s_all.at[pl.ds(c * CHUNK, CHUNK)]],
                            add=True)
            if do_hist:
                @pl.when(cid == 0)
                def _():
                    pltpu.sync_copy(onesb,
                                    hist_sh.at[bgi.at[pl.ds(c * CHUNK, CHUNK)]],
                                    add=True)

                @pl.when(cid == 1)
                def _():
                    pltpu.sync_copy(onesb,
                                    hist_sh.at[bsi.at[pl.ds(c * CHUNK, CHUNK)]],
                                    add=True)

        fire(0, rows0, sem0)

        def pair_body(p, carry):
            c0 = 2 * p
            c1 = 2 * p + 1
            fire(c1, rows1, sem1)
            wait_g(c0, rows0, sem0)
            scatter(c0, rows0)

            @pl.when(c1 + 1 < CHUNKS_PER_TILE)
            def _():
                fire(c1 + 1, rows0, sem0)

            wait_g(c1, rows1, sem1)
            scatter(c1, rows1)
            return carry

        lax.fori_loop(0, CHUNKS_PER_TILE // 2, pair_body, 0)
        plsc.subcore_barrier()

        def wb_body(i, c):
            off = r0 + i * B_ROWS
            pltpu.sync_copy(acc_sh.at[pl.ds(off, B_ROWS)], bounce)
            pltpu.sync_copy(bounce, out_ref.at[pl.ds(off, B_ROWS)])
            return c

        lax.fori_loop(0, H_SLICE // B_ROWS, wb_body, 0)

        @pl.when(wid == 0)
        def _():
            pltpu.sync_copy(acc_sh.at[pl.ds(H_TAIL_OFF, H_TAIL)],
                            bounce.at[pl.ds(0, H_TAIL)])
            pltpu.sync_copy(bounce.at[pl.ds(0, H_TAIL)],
                            out_ref.at[pl.ds(H_TAIL_OFF, H_TAIL)])

        lax.fori_loop(0, B_ROWS * (QCOL // 16), zb_body, 0)

    @pl.when(cid == 0)
    def _():
        round_q(0, tab0, out0, with_hist)
        round_q(1, tab1, out1, False)

    @pl.when(cid == 1)
    def _():
        round_q(0, tab2, out2, with_hist)
        round_q(1, tab3, out3, False)

    if with_hist:
        plsc.subcore_barrier()

        def _hist_out(h_ref):
            pltpu.sync_copy(hist_sh.at[pl.ds(r0, H_SLICE)], hbuf)
            pltpu.sync_copy(hbuf, h_ref.at[pl.ds(r0, H_SLICE)])

            @pl.when(wid == 0)
            def _():
                pltpu.sync_copy(hist_sh.at[pl.ds(H_TAIL_OFF, H_TAIL)],
                                hbuf.at[pl.ds(0, H_TAIL)])
                pltpu.sync_copy(hbuf.at[pl.ds(0, H_TAIL)],
                                h_ref.at[pl.ds(H_TAIL_OFF, H_TAIL)])

        @pl.when(cid == 0)
        def _():
            _hist_out(histA)

        @pl.when(cid == 1)
        def _():
            _hist_out(histB)


def _make_sc_pass(gather_by_edge, with_hist):
    mesh = plsc.VectorSubcoreMesh(core_axis_name="c", subcore_axis_name="s",
                                  num_cores=NUM_CORES, num_subcores=NUM_TILES)
    f32, i32 = jnp.float32, jnp.int32
    out_type = [jax.ShapeDtypeStruct((TWO_N, QCOL), f32) for _ in range(NQ)]
    scratch = [pltpu.VMEM((EDGES_PER_TILE,), i32),
               pltpu.VMEM((EDGES_PER_TILE,), i32),
               ]
    if with_hist:
        out_type += [jax.ShapeDtypeStruct((TWO_N,), f32),
                     jax.ShapeDtypeStruct((TWO_N,), f32)]
        scratch += [pltpu.VMEM((CHUNK,), f32),
                    pltpu.VMEM((H_SLICE,), f32)]
    scratch += [pltpu.VMEM((CHUNK, QCOL), f32),
                pltpu.VMEM((CHUNK, QCOL), f32)]
    scratch += [pltpu.VMEM((B_ROWS, QCOL), f32)]
    scratch += [pltpu.VMEM_SHARED((TWO_N, QCOL), f32)]
    if with_hist:
        scratch += [pltpu.VMEM_SHARED((TWO_N,), f32)]
    scratch += [pltpu.SemaphoreType.DMA, pltpu.SemaphoreType.DMA]

    body = functools.partial(_sc_pass_body, gather_by_edge, with_hist)
    return pl.kernel(body, out_type=out_type, mesh=mesh,
                     scratch_types=scratch,
                     compiler_params=pltpu.CompilerParams(
                         use_tc_tiling_on_sc=False))


_sc_pass1 = _make_sc_pass(gather_by_edge=False, with_hist=True)
_sc_pass2 = _make_sc_pass(gather_by_edge=True, with_hist=False)


def _xl_body(x_ref, W_ref, o0_ref, o1_ref, o2_ref, o3_ref):
    res = lax.dot_general(x_ref[...], W_ref[0],
                          (((1,), (1,)), ((), ())),
                          preferred_element_type=jnp.float32)
    o0_ref[0] = res[:, 0 * QCOL:1 * QCOL]
    o1_ref[0] = res[:, 1 * QCOL:2 * QCOL]
    o2_ref[0] = res[:, 2 * QCOL:3 * QCOL]
    o3_ref[0] = res[:, 3 * QCOL:4 * QCOL]


def _xl_quarters(x, W01):
    nb = 25
    blk = N // nb
    outs = pl.pallas_call(
        _xl_body,
        grid=(nb, 2),
        in_specs=[pl.BlockSpec((blk, HID), lambda i, t: (i, 0)),
                  pl.BlockSpec((1, HID, HID), lambda i, t: (t, 0, 0))],
        out_specs=[pl.BlockSpec((1, blk, QCOL), lambda i, t: (t, i, 0))
                   for _ in range(NQ)],
        out_shape=[jax.ShapeDtypeStruct((2, N, QCOL), jnp.float32)
                   for _ in range(NQ)],
    )(x, W01)
    return [o.reshape(TWO_N, QCOL) for o in outs]


def _scale_body(e0, e1, e2, e3, bd_ref, dd_ref,
                o0, o1, o2, o3, dinv_ref):
    bd = bd_ref[0, 0, :]
    binv = jnp.where(bd > 0, 1.0 / bd, 0.0).reshape(-1, 1)
    o0[...] = e0[...] * binv
    o1[...] = e1[...] * binv
    o2[...] = e2[...] * binv
    o3[...] = e3[...] * binv
    dd = dd_ref[0, 0, :]
    dinv_ref[0, 0, :] = jnp.where(dd > 0, 1.0 / dd, 0.0)


def _scale_ef(efq, Bd, Dd):
    nb = 50
    blk = TWO_N // nb
    res = pl.pallas_call(
        _scale_body,
        grid=(nb,),
        in_specs=[pl.BlockSpec((blk, QCOL), lambda i: (i, 0))
                  for _ in range(NQ)]
        + [pl.BlockSpec((1, 1, blk), lambda i: (i, 0, 0)),
           pl.BlockSpec((1, 1, blk), lambda i: (i, 0, 0))],
        out_specs=[pl.BlockSpec((blk, QCOL), lambda i: (i, 0))
                   for _ in range(NQ)]
        + [pl.BlockSpec((1, 1, blk), lambda i: (i, 0, 0))],
        out_shape=[jax.ShapeDtypeStruct((TWO_N, QCOL), jnp.float32)
                   for _ in range(NQ)]
        + [jax.ShapeDtypeStruct((nb, 1, blk), jnp.float32)],
    )(*efq, Bd.reshape(nb, 1, blk), Dd.reshape(nb, 1, blk))
    return res[:NQ], res[NQ]


def _final_body(q00, q01, q02, q03, q10, q11, q12, q13, d0, d1, hp_ref,
                Wm0T, Wm1T, bh, WihT, bih, WhhT, bhh, WroT, bro,
                hn_ref, pr_ref):
    dinv0 = d0[0, 0, :].reshape(-1, 1)
    dinv1 = d1[0, 0, :].reshape(-1, 1)
    x0 = jnp.concatenate([q00[...], q01[...], q02[...], q03[...]],
                         axis=1) * dinv0
    x1 = jnp.concatenate([q10[...], q11[...], q12[...], q13[...]],
                         axis=1) * dinv1
    pre = (jnp.dot(x0, Wm0T[...], preferred_element_type=jnp.float32)
           + jnp.dot(x1, Wm1T[...], preferred_element_type=jnp.float32)
           + bh[...])
    h = jnp.maximum(pre, 0.0)
    gi = jnp.dot(h, WihT[...], preferred_element_type=jnp.float32) + bih[...]
    hp = hp_ref[...]
    gh = jnp.dot(hp, WhhT[...], preferred_element_type=jnp.float32) + bhh[...]
    r = jax.nn.sigmoid(gi[:, :HID] + gh[:, :HID])
    z = jax.nn.sigmoid(gi[:, HID:2 * HID] + gh[:, HID:2 * HID])
    n = jnp.tanh(gi[:, 2 * HID:] + r * gh[:, 2 * HID:])
    hn = (1.0 - z) * n + z * hp
    hn_ref[...] = hn
    pr_ref[...] = jnp.dot(hn, WroT[...], preferred_element_type=jnp.float32) + bro[...]


def _final(outq, Dinv3, h_prev, Wm0T, Wm1T, bias_h,
           WihT, b_ih, WhhT, b_hh, WroT, b_ro):
    nb = 25
    blk = N // nb
    full = lambda shape: pl.BlockSpec(shape, lambda i: (0,) * len(shape))
    return pl.pallas_call(
        _final_body,
        grid=(nb,),
        in_specs=[pl.BlockSpec((blk, QCOL), lambda i: (i, 0))
                  for _ in range(NQ)]
        + [pl.BlockSpec((blk, QCOL), lambda i: (nb + i, 0))
           for _ in range(NQ)]
        + [pl.BlockSpec((1, 1, blk), lambda i: (i, 0, 0)),
           pl.BlockSpec((1, 1, blk), lambda i: (nb + i, 0, 0)),
           pl.BlockSpec((blk, HID), lambda i: (i, 0)),
           full((HID, HID)), full((HID, HID)), full((1, HID)),
           full((HID, 3 * HID)), full((1, 3 * HID)),
           full((HID, 3 * HID)), full((1, 3 * HID)),
           full((HID, OUT_DIM)), full((1, OUT_DIM))],
        out_specs=[pl.BlockSpec((blk, HID), lambda i: (i, 0)),
                   pl.BlockSpec((blk, OUT_DIM), lambda i: (i, 0))],
        out_shape=[jax.ShapeDtypeStruct((N, HID), jnp.float32),
                   jax.ShapeDtypeStruct((N, OUT_DIM), jnp.float32)],
    )(*outq, *outq, Dinv3, Dinv3, h_prev, Wm0T, Wm1T, bias_h,
      WihT, b_ih, WhhT, b_hh, WroT, b_ro)


def kernel(x, edge_index, edge_attr, h_prev, W0, b0, W1, b1, W_mix, b_mix,
           W_ih, W_hh, b_ih, b_hh, W_ro, b_ro):
    node_idx = edge_index[0]
    hedge_idx = edge_index[1]

    xlq = _xl_quarters(x, jnp.stack([W0, W1]))

    res1 = _sc_pass1(node_idx, hedge_idx, edge_attr, *xlq)
    efq, Dd, Bd = res1[:NQ], res1[NQ], res1[NQ + 1]

    efsq, Dinv3 = _scale_ef(efq, Bd, Dd)

    outq = _sc_pass2(node_idx, hedge_idx, edge_attr, *efsq)

    bias_h = (b0 @ W_mix[:, :HID].T + b1 @ W_mix[:, HID:].T + b_mix)
    h_next, pred = _final(
        outq, Dinv3, h_prev,
        W_mix[:, :HID].T, W_mix[:, HID:].T, bias_h.reshape(1, HID),
        W_ih.T, b_ih.reshape(1, 3 * HID), W_hh.T, b_hh.reshape(1, 3 * HID),
        W_ro.T, b_ro.reshape(1, OUT_DIM))
    return (h_next, pred[:, :3])


# speedup vs baseline: 30.4813x; 1.0976x over previous
import functools

import jax
import jax.numpy as jnp
from jax import lax
from jax.experimental import pallas as pl
from jax.experimental.pallas import tpu as pltpu
from jax.experimental.pallas import tpu_sc as plsc

N = 10000
E = 320000
TWO_N = 2 * N
HID = 128
QCOL = 32
NQ = HID // QCOL
OUT_DIM = 64

NUM_CORES = 2
NUM_TILES = 16
CHUNK = 400
VECS = CHUNK // 16
EDGES_PER_TILE = E // NUM_TILES
CHUNKS_PER_TILE = EDGES_PER_TILE // CHUNK
H_SLICE = 1248
H_TAIL_OFF = H_SLICE * NUM_TILES
H_TAIL = TWO_N - H_TAIL_OFF
B_ROWS = 416


def _sc_pass_body(gather_by_edge, with_hist, *refs):
    if with_hist:
        (ni_hbm, ee_hbm, at_hbm, tab0, tab1, tab2, tab3,
         out0, out1, out2, out3, histA, histB,
         bgi, bsi, onesb, hbuf, rows0, rows1, bounce,
         acc_sh, hist_sh, sem0, sem1) = refs
    else:
        (ni_hbm, ee_hbm, at_hbm, tab0, tab1, tab2, tab3,
         out0, out1, out2, out3,
         bgi, bsi, rows0, rows1, bounce, acc_sh, sem0, sem1) = refs

    cid = lax.axis_index("c")
    wid = lax.axis_index("s")
    r0 = wid * H_SLICE
    ebase = wid * EDGES_PER_TILE

    d0 = pltpu.async_copy(ni_hbm.at[pl.ds(ebase, EDGES_PER_TILE)], bgi, sem0)
    d1 = pltpu.async_copy(at_hbm.at[pl.ds(ebase, EDGES_PER_TILE)], bsi, sem1)

    zeros16 = jnp.zeros((16,), jnp.float32)

    def zb_body(i, c):
        r = i // 2
        j = (i % 2) * 16
        bounce[r, pl.ds(j, 16)] = zeros16
        return c

    lax.fori_loop(0, B_ROWS * (QCOL // 16), zb_body, 0)

    if with_hist:
        def zh_body(i, c):
            hbuf[pl.ds(i * 16, 16)] = zeros16
            return c

        lax.fori_loop(0, H_SLICE // 16, zh_body, 0)
        pltpu.sync_copy(hbuf, hist_sh.at[pl.ds(r0, H_SLICE)])

        @pl.when(wid == 0)
        def _():
            pltpu.sync_copy(hbuf.at[pl.ds(0, H_TAIL)],
                            hist_sh.at[pl.ds(H_TAIL_OFF, H_TAIL)])

        ones16 = jnp.ones((16,), jnp.float32)

        def ob_body(i, c):
            onesb[pl.ds(i * 16, 16)] = ones16
            return c

        lax.fori_loop(0, VECS, ob_body, 0)

    d0.wait()
    d1.wait()

    def ix1_body(i, c):
        s = i * 16
        bgi[pl.ds(s, 16)] = bgi[pl.ds(s, 16)] + bsi[pl.ds(s, 16)] * N
        return c

    lax.fori_loop(0, EDGES_PER_TILE // 16, ix1_body, 0)

    pltpu.async_copy(ee_hbm.at[pl.ds(ebase, EDGES_PER_TILE)], bsi, sem0).wait()

    def ix2_body(i, c):
        s = i * 16
        g = bgi[pl.ds(s, 16)]
        off = jnp.where(g >= N, N, 0).astype(jnp.int32)
        bsi[pl.ds(s, 16)] = bsi[pl.ds(s, 16)] + off
        return c

    lax.fori_loop(0, EDGES_PER_TILE // 16, ix2_body, 0)

    g_all = bsi if gather_by_edge else bgi
    s_all = bgi if gather_by_edge else bsi

    def round_q(tab, out_ref, do_hist):
        def zacc_body(i, c):
            pltpu.sync_copy(bounce, acc_sh.at[pl.ds(r0 + i * B_ROWS, B_ROWS)])
            return c

        lax.fori_loop(0, H_SLICE // B_ROWS, zacc_body, 0)

        @pl.when(wid == 0)
        def _():
            pltpu.sync_copy(bounce.at[pl.ds(0, H_TAIL)],
                            acc_sh.at[pl.ds(H_TAIL_OFF, H_TAIL)])

        plsc.subcore_barrier()

        def fire(c, rows, sem):
            pltpu.async_copy(tab.at[g_all.at[pl.ds(c * CHUNK, CHUNK)]],
                             rows, sem)

        def wait_g(c, rows, sem):
            pltpu.make_async_copy(tab.at[g_all.at[pl.ds(c * CHUNK, CHUNK)]],
                                  rows, sem).wait()

        def scatter(c, rows):
            pltpu.sync_copy(rows, acc_sh.at[s_all.at[pl.ds(c * CHUNK, CHUNK)]],
                            add=True)
            if do_hist:
                @pl.when(cid == 0)
                def _():
                    pltpu.sync_copy(onesb,
                                    hist_sh.at[bgi.at[pl.ds(c * CHUNK, CHUNK)]],
                                    add=True)

                @pl.when(cid == 1)
                def _():
                    pltpu.sync_copy(onesb,
                                    hist_sh.at[bsi.at[pl.ds(c * CHUNK, CHUNK)]],
                                    add=True)

        fire(0, rows0, sem0)

        def pair_body(p, carry):
            c0 = 2 * p
            c1 = 2 * p + 1
            fire(c1, rows1, sem1)
            wait_g(c0, rows0, sem0)
            scatter(c0, rows0)

            @pl.when(c1 + 1 < CHUNKS_PER_TILE)
            def _():
                fire(c1 + 1, rows0, sem0)

            wait_g(c1, rows1, sem1)
            scatter(c1, rows1)
            return carry

        lax.fori_loop(0, CHUNKS_PER_TILE // 2, pair_body, 0)
        plsc.subcore_barrier()

        def wb_body(i, c):
            off = r0 + i * B_ROWS
            pltpu.sync_copy(acc_sh.at[pl.ds(off, B_ROWS)], bounce)
            pltpu.sync_copy(bounce, out_ref.at[pl.ds(off, B_ROWS)])
            return c

        lax.fori_loop(0, H_SLICE // B_ROWS, wb_body, 0)

        @pl.when(wid == 0)
        def _():
            pltpu.sync_copy(acc_sh.at[pl.ds(H_TAIL_OFF, H_TAIL)],
                            bounce.at[pl.ds(0, H_TAIL)])
            pltpu.sync_copy(bounce.at[pl.ds(0, H_TAIL)],
                            out_ref.at[pl.ds(H_TAIL_OFF, H_TAIL)])

        lax.fori_loop(0, B_ROWS * (QCOL // 16), zb_body, 0)

    @pl.when(cid == 0)
    def _():
        round_q(tab0, out0, with_hist)
        round_q(tab1, out1, False)

    @pl.when(cid == 1)
    def _():
        round_q(tab2, out2, with_hist)
        round_q(tab3, out3, False)

    if with_hist:
        plsc.subcore_barrier()

        def _hist_out(h_ref):
            pltpu.sync_copy(hist_sh.at[pl.ds(r0, H_SLICE)], hbuf)
            pltpu.sync_copy(hbuf, h_ref.at[pl.ds(r0, H_SLICE)])

            @pl.when(wid == 0)
            def _():
                pltpu.sync_copy(hist_sh.at[pl.ds(H_TAIL_OFF, H_TAIL)],
                                hbuf.at[pl.ds(0, H_TAIL)])
                pltpu.sync_copy(hbuf.at[pl.ds(0, H_TAIL)],
                                h_ref.at[pl.ds(H_TAIL_OFF, H_TAIL)])

        @pl.when(cid == 0)
        def _():
            _hist_out(histA)

        @pl.when(cid == 1)
        def _():
            _hist_out(histB)


def _make_sc_pass(gather_by_edge, with_hist):
    mesh = plsc.VectorSubcoreMesh(core_axis_name="c", subcore_axis_name="s",
                                  num_cores=NUM_CORES, num_subcores=NUM_TILES)
    f32, i32 = jnp.float32, jnp.int32
    out_type = [jax.ShapeDtypeStruct((TWO_N, QCOL), f32) for _ in range(NQ)]
    scratch = [pltpu.VMEM((EDGES_PER_TILE,), i32),
               pltpu.VMEM((EDGES_PER_TILE,), i32),
               ]
    if with_hist:
        out_type += [jax.ShapeDtypeStruct((TWO_N,), f32),
                     jax.ShapeDtypeStruct((TWO_N,), f32)]
        scratch += [pltpu.VMEM((CHUNK,), f32),
                    pltpu.VMEM((H_SLICE,), f32)]
    scratch += [pltpu.VMEM((CHUNK, QCOL), f32),
                pltpu.VMEM((CHUNK, QCOL), f32)]
    scratch += [pltpu.VMEM((B_ROWS, QCOL), f32)]
    scratch += [pltpu.VMEM_SHARED((TWO_N, QCOL), f32)]
    if with_hist:
        scratch += [pltpu.VMEM_SHARED((TWO_N,), f32)]
    scratch += [pltpu.SemaphoreType.DMA, pltpu.SemaphoreType.DMA]

    body = functools.partial(_sc_pass_body, gather_by_edge, with_hist)
    return pl.kernel(body, out_type=out_type, mesh=mesh,
                     scratch_types=scratch,
                     compiler_params=pltpu.CompilerParams(
                         use_tc_tiling_on_sc=False))


_sc_pass1 = _make_sc_pass(gather_by_edge=False, with_hist=True)
_sc_pass2 = _make_sc_pass(gather_by_edge=True, with_hist=False)


def _xl_body(x_ref, W_ref, o0_ref, o1_ref, o2_ref, o3_ref):
    res = lax.dot_general(x_ref[...], W_ref[0],
                          (((1,), (1,)), ((), ())),
                          preferred_element_type=jnp.float32)
    o0_ref[...] = res[:, 0 * QCOL:1 * QCOL]
    o1_ref[...] = res[:, 1 * QCOL:2 * QCOL]
    o2_ref[...] = res[:, 2 * QCOL:3 * QCOL]
    o3_ref[...] = res[:, 3 * QCOL:4 * QCOL]


def _xl_quarters(x, W01):
    nb = 5
    blk = N // nb
    return pl.pallas_call(
        _xl_body,
        grid=(nb, 2),
        in_specs=[pl.BlockSpec((blk, HID), lambda i, t: (i, 0)),
                  pl.BlockSpec((1, HID, HID), lambda i, t: (t, 0, 0))],
        out_specs=[pl.BlockSpec((blk, QCOL), lambda i, t: (t * 5 + i, 0))
                   for _ in range(NQ)],
        out_shape=[jax.ShapeDtypeStruct((TWO_N, QCOL), jnp.float32)
                   for _ in range(NQ)],
    )(x, W01)


def _scale_body(e0, e1, e2, e3, bd_ref, dd_ref,
                o0, o1, o2, o3, dinv_ref):
    bd = bd_ref[0, 0, :]
    binv = jnp.where(bd > 0, 1.0 / bd, 0.0).reshape(-1, 1)
    o0[...] = e0[...] * binv
    o1[...] = e1[...] * binv
    o2[...] = e2[...] * binv
    o3[...] = e3[...] * binv
    dd = dd_ref[0, 0, :]
    dinv_ref[0, 0, :] = jnp.where(dd > 0, 1.0 / dd, 0.0)


def _scale_ef(efq, Bd, Dd):
    nb = 20
    blk = TWO_N // nb
    res = pl.pallas_call(
        _scale_body,
        grid=(nb,),
        in_specs=[pl.BlockSpec((blk, QCOL), lambda i: (i, 0))
                  for _ in range(NQ)]
        + [pl.BlockSpec((1, 1, blk), lambda i: (i, 0, 0)),
           pl.BlockSpec((1, 1, blk), lambda i: (i, 0, 0))],
        out_specs=[pl.BlockSpec((blk, QCOL), lambda i: (i, 0))
                   for _ in range(NQ)]
        + [pl.BlockSpec((1, 1, blk), lambda i: (i, 0, 0))],
        out_shape=[jax.ShapeDtypeStruct((TWO_N, QCOL), jnp.float32)
                   for _ in range(NQ)]
        + [jax.ShapeDtypeStruct((nb, 1, blk), jnp.float32)],
    )(*efq, Bd.reshape(nb, 1, blk), Dd.reshape(nb, 1, blk))
    return res[:NQ], res[NQ]


def _final_body(q00, q01, q02, q03, q10, q11, q12, q13, d0, d1, hp_ref,
                Wm0T, Wm1T, bh, WihT, bih, WhhT, bhh, WroT, bro,
                hn_ref, pr_ref):
    x0 = jnp.concatenate([q00[...], q01[...], q02[...], q03[...]],
                         axis=1) * d0[0, 0, :].reshape(-1, 1)
    x1 = jnp.concatenate([q10[...], q11[...], q12[...], q13[...]],
                         axis=1) * d1[0, 0, :].reshape(-1, 1)
    pre = (jnp.dot(x0, Wm0T[...], preferred_element_type=jnp.float32)
           + jnp.dot(x1, Wm1T[...], preferred_element_type=jnp.float32)
           + bh[...])
    h = jnp.maximum(pre, 0.0)
    gi = jnp.dot(h, WihT[...], preferred_element_type=jnp.float32) + bih[...]
    hp = hp_ref[...]
    gh = jnp.dot(hp, WhhT[...], preferred_element_type=jnp.float32) + bhh[...]
    r = jax.nn.sigmoid(gi[:, :HID] + gh[:, :HID])
    z = jax.nn.sigmoid(gi[:, HID:2 * HID] + gh[:, HID:2 * HID])
    n = jnp.tanh(gi[:, 2 * HID:] + r * gh[:, 2 * HID:])
    hn = (1.0 - z) * n + z * hp
    hn_ref[...] = hn
    pr_ref[...] = jnp.dot(hn, WroT[...], preferred_element_type=jnp.float32) + bro[...]


def _final(outq, Dinv3, h_prev, Wm0T, Wm1T, bias_h,
           WihT, b_ih, WhhT, b_hh, WroT, b_ro):
    nb = 10
    blk = N // nb
    full = lambda shape: pl.BlockSpec(shape, lambda i: (0,) * len(shape))
    return pl.pallas_call(
        _final_body,
        grid=(nb,),
        in_specs=[pl.BlockSpec((blk, QCOL), lambda i: (i, 0))
                  for _ in range(NQ)]
        + [pl.BlockSpec((blk, QCOL), lambda i: (nb + i, 0))
           for _ in range(NQ)]
        + [pl.BlockSpec((1, 1, blk), lambda i: (i, 0, 0)),
           pl.BlockSpec((1, 1, blk), lambda i: (nb + i, 0, 0)),
           pl.BlockSpec((blk, HID), lambda i: (i, 0)),
           full((HID, HID)), full((HID, HID)), full((1, HID)),
           full((HID, 3 * HID)), full((1, 3 * HID)),
           full((HID, 3 * HID)), full((1, 3 * HID)),
           full((HID, OUT_DIM)), full((1, OUT_DIM))],
        out_specs=[pl.BlockSpec((blk, HID), lambda i: (i, 0)),
                   pl.BlockSpec((blk, OUT_DIM), lambda i: (i, 0))],
        out_shape=[jax.ShapeDtypeStruct((N, HID), jnp.float32),
                   jax.ShapeDtypeStruct((N, OUT_DIM), jnp.float32)],
    )(*outq, *outq, Dinv3, Dinv3, h_prev, Wm0T, Wm1T, bias_h,
      WihT, b_ih, WhhT, b_hh, WroT, b_ro)


def kernel(x, edge_index, edge_attr, h_prev, W0, b0, W1, b1, W_mix, b_mix,
           W_ih, W_hh, b_ih, b_hh, W_ro, b_ro):
    node_idx = edge_index[0]
    hedge_idx = edge_index[1]

    xlq = _xl_quarters(x, jnp.stack([W0, W1]))

    res1 = _sc_pass1(node_idx, hedge_idx, edge_attr, *xlq)
    efq, Dd, Bd = res1[:NQ], res1[NQ], res1[NQ + 1]

    efsq, Dinv = _scale_ef(efq, Bd, Dd)

    outq = _sc_pass2(node_idx, hedge_idx, edge_attr, *efsq)

    bias_h = (b0 @ W_mix[:, :HID].T + b1 @ W_mix[:, HID:].T + b_mix)
    h_next, pred = _final(
        outq, Dinv, h_prev,
        W_mix[:, :HID].T, W_mix[:, HID:].T, bias_h.reshape(1, HID),
        W_ih.T, b_ih.reshape(1, 3 * HID), W_hh.T, b_hh.reshape(1, 3 * HID),
        W_ro.T, b_ro.reshape(1, OUT_DIM))
    return (h_next, pred[:, :3])


# speedup vs baseline: 35.6190x; 1.1686x over previous
import functools

import jax
import jax.numpy as jnp
from jax import lax
from jax.experimental import pallas as pl
from jax.experimental.pallas import tpu as pltpu
from jax.experimental.pallas import tpu_sc as plsc

N = 10000
E = 320000
TWO_N = 2 * N
HID = 128
QCOL = 32
NQ = HID // QCOL
OUT_DIM = 64

NUM_CORES = 2
NUM_TILES = 16
CHUNK = 400
VECS = CHUNK // 16
EDGES_PER_TILE = E // NUM_TILES
CHUNKS_PER_TILE = EDGES_PER_TILE // CHUNK
H_SLICE = 1248
H_TAIL_OFF = H_SLICE * NUM_TILES
H_TAIL = TWO_N - H_TAIL_OFF
B_ROWS = 416


def _sc_pass_body(gather_by_edge, *refs):
    (ni_hbm, ee_hbm, at_hbm, tab0, tab1, tab2, tab3,
     out0, out1, out2, out3,
     bgi, bsi, onesb, hbuf, htail, rows0, rows1, bounce,
     acc_sh, hist_sh, sem0, sem1) = refs

    cid = lax.axis_index("c")
    wid = lax.axis_index("s")
    r0 = wid * H_SLICE
    ebase = wid * EDGES_PER_TILE

    d0 = pltpu.async_copy(ni_hbm.at[pl.ds(ebase, EDGES_PER_TILE)], bgi, sem0)
    d1 = pltpu.async_copy(at_hbm.at[pl.ds(ebase, EDGES_PER_TILE)], bsi, sem1)

    zeros16 = jnp.zeros((16,), jnp.float32)

    def zb_body(i, c):
        r = i // 2
        j = (i % 2) * 16
        bounce[r, pl.ds(j, 16)] = zeros16
        return c

    lax.fori_loop(0, B_ROWS * (QCOL // 16), zb_body, 0)

    def zh_body(i, c):
        hbuf[pl.ds(i * 16, 16)] = zeros16
        return c

    lax.fori_loop(0, H_SLICE // 16, zh_body, 0)
    pltpu.sync_copy(hbuf, hist_sh.at[pl.ds(r0, H_SLICE)])

    @pl.when(wid == 0)
    def _():
        pltpu.sync_copy(hbuf.at[pl.ds(0, H_TAIL)],
                        hist_sh.at[pl.ds(H_TAIL_OFF, H_TAIL)])

    ones16 = jnp.ones((16,), jnp.float32)

    def ob_body(i, c):
        onesb[pl.ds(i * 16, 16)] = ones16
        return c

    lax.fori_loop(0, VECS, ob_body, 0)

    d0.wait()
    d1.wait()

    def ix1_body(i, c):
        s = i * 16
        bgi[pl.ds(s, 16)] = bgi[pl.ds(s, 16)] + bsi[pl.ds(s, 16)] * N
        return c

    lax.fori_loop(0, EDGES_PER_TILE // 16, ix1_body, 0)

    pltpu.async_copy(ee_hbm.at[pl.ds(ebase, EDGES_PER_TILE)], bsi, sem0).wait()

    def ix2_body(i, c):
        s = i * 16
        g = bgi[pl.ds(s, 16)]
        off = jnp.where(g >= N, N, 0).astype(jnp.int32)
        bsi[pl.ds(s, 16)] = bsi[pl.ds(s, 16)] + off
        return c

    lax.fori_loop(0, EDGES_PER_TILE // 16, ix2_body, 0)

    g_all = bsi if gather_by_edge else bgi
    s_all = bgi if gather_by_edge else bsi

    def round_q(tab, out_ref, do_hist):
        def zacc_body(i, c):
            pltpu.sync_copy(bounce, acc_sh.at[pl.ds(r0 + i * B_ROWS, B_ROWS)])
            return c

        lax.fori_loop(0, H_SLICE // B_ROWS, zacc_body, 0)

        @pl.when(wid == 0)
        def _():
            pltpu.sync_copy(bounce.at[pl.ds(0, H_TAIL)],
                            acc_sh.at[pl.ds(H_TAIL_OFF, H_TAIL)])

        plsc.subcore_barrier()

        def fire(c, rows, sem):
            pltpu.async_copy(tab.at[g_all.at[pl.ds(c * CHUNK, CHUNK)]],
                             rows, sem)

        def wait_g(c, rows, sem):
            pltpu.make_async_copy(tab.at[g_all.at[pl.ds(c * CHUNK, CHUNK)]],
                                  rows, sem).wait()

        def scatter(c, rows):
            pltpu.sync_copy(rows, acc_sh.at[s_all.at[pl.ds(c * CHUNK, CHUNK)]],
                            add=True)
            if do_hist:
                pltpu.sync_copy(onesb,
                                hist_sh.at[s_all.at[pl.ds(c * CHUNK, CHUNK)]],
                                add=True)

        fire(0, rows0, sem0)

        def pair_body(p, carry):
            c0 = 2 * p
            c1 = 2 * p + 1
            fire(c1, rows1, sem1)
            wait_g(c0, rows0, sem0)
            scatter(c0, rows0)

            @pl.when(c1 + 1 < CHUNKS_PER_TILE)
            def _():
                fire(c1 + 1, rows0, sem0)

            wait_g(c1, rows1, sem1)
            scatter(c1, rows1)
            return carry

        lax.fori_loop(0, CHUNKS_PER_TILE // 2, pair_body, 0)
        plsc.subcore_barrier()

        if do_hist:
            pltpu.sync_copy(hist_sh.at[pl.ds(r0, H_SLICE)], hbuf)

            @pl.when(wid == 0)
            def _():
                pltpu.sync_copy(hist_sh.at[pl.ds(H_TAIL_OFF, H_TAIL)], htail)

        def wb_body(i, c):
            off = r0 + i * B_ROWS
            pltpu.sync_copy(acc_sh.at[pl.ds(off, B_ROWS)], bounce)

            def sc_body(g, c2):
                base = g * 16
                dvec = hbuf[pl.ds(i * B_ROWS + base, 16)]
                dinv = jnp.where(dvec > 0, 1.0 / dvec, 0.0)
                for l in range(16):
                    s = dinv[l]
                    bounce[base + l, pl.ds(0, 16)] = (
                        bounce[base + l, pl.ds(0, 16)] * s)
                    bounce[base + l, pl.ds(16, 16)] = (
                        bounce[base + l, pl.ds(16, 16)] * s)
                return c2

            lax.fori_loop(0, B_ROWS // 16, sc_body, 0)
            pltpu.sync_copy(bounce, out_ref.at[pl.ds(off, B_ROWS)])
            return c

        lax.fori_loop(0, H_SLICE // B_ROWS, wb_body, 0)

        @pl.when(wid == 0)
        def _():
            pltpu.sync_copy(acc_sh.at[pl.ds(H_TAIL_OFF, H_TAIL)],
                            bounce.at[pl.ds(0, H_TAIL)])

            def sct_body(g, c2):
                base = g * 16
                dvec = htail[pl.ds(base, 16)]
                dinv = jnp.where(dvec > 0, 1.0 / dvec, 0.0)
                for l in range(16):
                    s = dinv[l]
                    bounce[base + l, pl.ds(0, 16)] = (
                        bounce[base + l, pl.ds(0, 16)] * s)
                    bounce[base + l, pl.ds(16, 16)] = (
                        bounce[base + l, pl.ds(16, 16)] * s)
                return c2

            lax.fori_loop(0, H_TAIL // 16, sct_body, 0)
            pltpu.sync_copy(bounce.at[pl.ds(0, H_TAIL)],
                            out_ref.at[pl.ds(H_TAIL_OFF, H_TAIL)])

        lax.fori_loop(0, B_ROWS * (QCOL // 16), zb_body, 0)

    @pl.when(cid == 0)
    def _():
        round_q(tab0, out0, True)
        round_q(tab1, out1, False)

    @pl.when(cid == 1)
    def _():
        round_q(tab2, out2, True)
        round_q(tab3, out3, False)


def _make_sc_pass(gather_by_edge):
    mesh = plsc.VectorSubcoreMesh(core_axis_name="c", subcore_axis_name="s",
                                  num_cores=NUM_CORES, num_subcores=NUM_TILES)
    f32, i32 = jnp.float32, jnp.int32
    out_type = [jax.ShapeDtypeStruct((TWO_N, QCOL), f32) for _ in range(NQ)]
    scratch = [pltpu.VMEM((EDGES_PER_TILE,), i32),
               pltpu.VMEM((EDGES_PER_TILE,), i32),
               pltpu.VMEM((CHUNK,), f32),
               pltpu.VMEM((H_SLICE,), f32),
               pltpu.VMEM((H_TAIL,), f32),
               pltpu.VMEM((CHUNK, QCOL), f32),
               pltpu.VMEM((CHUNK, QCOL), f32),
               pltpu.VMEM((B_ROWS, QCOL), f32),
               pltpu.VMEM_SHARED((TWO_N, QCOL), f32),
               pltpu.VMEM_SHARED((TWO_N,), f32),
               pltpu.SemaphoreType.DMA, pltpu.SemaphoreType.DMA]

    body = functools.partial(_sc_pass_body, gather_by_edge)
    return pl.kernel(body, out_type=out_type, mesh=mesh,
                     scratch_types=scratch,
                     compiler_params=pltpu.CompilerParams(
                         use_tc_tiling_on_sc=False))


_sc_pass1 = _make_sc_pass(gather_by_edge=False)
_sc_pass2 = _make_sc_pass(gather_by_edge=True)


def _xl_body(x_ref, W_ref, o0_ref, o1_ref, o2_ref, o3_ref):
    res = lax.dot_general(x_ref[...], W_ref[0],
                          (((1,), (1,)), ((), ())),
                          preferred_element_type=jnp.float32)
    o0_ref[...] = res[:, 0 * QCOL:1 * QCOL]
    o1_ref[...] = res[:, 1 * QCOL:2 * QCOL]
    o2_ref[...] = res[:, 2 * QCOL:3 * QCOL]
    o3_ref[...] = res[:, 3 * QCOL:4 * QCOL]


def _xl_quarters(x, W01):
    nb = 5
    blk = N // nb
    return pl.pallas_call(
        _xl_body,
        grid=(nb, 2),
        in_specs=[pl.BlockSpec((blk, HID), lambda i, t: (i, 0)),
                  pl.BlockSpec((1, HID, HID), lambda i, t: (t, 0, 0))],
        out_specs=[pl.BlockSpec((blk, QCOL), lambda i, t: (t * 5 + i, 0))
                   for _ in range(NQ)],
        out_shape=[jax.ShapeDtypeStruct((TWO_N, QCOL), jnp.float32)
                   for _ in range(NQ)],
    )(x, W01)


def _final_body(q00, q01, q02, q03, q10, q11, q12, q13, hp_ref,
                Wm0T, Wm1T, bh, WihT, bih, WhhT, bhh, WroT, bro,
                hn_ref, pr_ref):
    x0 = jnp.concatenate([q00[...], q01[...], q02[...], q03[...]], axis=1)
    x1 = jnp.concatenate([q10[...], q11[...], q12[...], q13[...]], axis=1)
    pre = (jnp.dot(x0, Wm0T[...], preferred_element_type=jnp.float32)
           + jnp.dot(x1, Wm1T[...], preferred_element_type=jnp.float32)
           + bh[...])
    h = jnp.maximum(pre, 0.0)
    gi = jnp.dot(h, WihT[...], preferred_element_type=jnp.float32) + bih[...]
    hp = hp_ref[...]
    gh = jnp.dot(hp, WhhT[...], preferred_element_type=jnp.float32) + bhh[...]
    r = jax.nn.sigmoid(gi[:, :HID] + gh[:, :HID])
    z = jax.nn.sigmoid(gi[:, HID:2 * HID] + gh[:, HID:2 * HID])
    n = jnp.tanh(gi[:, 2 * HID:] + r * gh[:, 2 * HID:])
    hn = (1.0 - z) * n + z * hp
    hn_ref[...] = hn
    pr_ref[...] = jnp.dot(hn, WroT[...], preferred_element_type=jnp.float32) + bro[...]


def _final(outq, h_prev, Wm0T, Wm1T, bias_h,
           WihT, b_ih, WhhT, b_hh, WroT, b_ro):
    nb = 10
    blk = N // nb
    full = lambda shape: pl.BlockSpec(shape, lambda i: (0,) * len(shape))
    return pl.pallas_call(
        _final_body,
        grid=(nb,),
        in_specs=[pl.BlockSpec((blk, QCOL), lambda i: (i, 0))
                  for _ in range(NQ)]
        + [pl.BlockSpec((blk, QCOL), lambda i: (nb + i, 0))
           for _ in range(NQ)]
        + [pl.BlockSpec((blk, HID), lambda i: (i, 0)),
           full((HID, HID)), full((HID, HID)), full((1, HID)),
           full((HID, 3 * HID)), full((1, 3 * HID)),
           full((HID, 3 * HID)), full((1, 3 * HID)),
           full((HID, OUT_DIM)), full((1, OUT_DIM))],
        out_specs=[pl.BlockSpec((blk, HID), lambda i: (i, 0)),
                   pl.BlockSpec((blk, OUT_DIM), lambda i: (i, 0))],
        out_shape=[jax.ShapeDtypeStruct((N, HID), jnp.float32),
                   jax.ShapeDtypeStruct((N, OUT_DIM), jnp.float32)],
    )(*outq, *outq, h_prev, Wm0T, Wm1T, bias_h,
      WihT, b_ih, WhhT, b_hh, WroT, b_ro)


def kernel(x, edge_index, edge_attr, h_prev, W0, b0, W1, b1, W_mix, b_mix,
           W_ih, W_hh, b_ih, b_hh, W_ro, b_ro):
    node_idx = edge_index[0]
    hedge_idx = edge_index[1]

    xlq = _xl_quarters(x, jnp.stack([W0, W1]))

    efq = _sc_pass1(node_idx, hedge_idx, edge_attr, *xlq)

    outq = _sc_pass2(node_idx, hedge_idx, edge_attr, *efq)

    bias_h = (b0 @ W_mix[:, :HID].T + b1 @ W_mix[:, HID:].T + b_mix)
    h_next, pred = _final(
        outq, h_prev,
        W_mix[:, :HID].T, W_mix[:, HID:].T, bias_h.reshape(1, HID),
        W_ih.T, b_ih.reshape(1, 3 * HID), W_hh.T, b_hh.reshape(1, 3 * HID),
        W_ro.T, b_ro.reshape(1, OUT_DIM))
    return (h_next, pred[:, :3])


# speedup vs baseline: 35.7200x; 1.0028x over previous
import functools

import jax
import jax.numpy as jnp
from jax import lax
from jax.experimental import pallas as pl
from jax.experimental.pallas import tpu as pltpu
from jax.experimental.pallas import tpu_sc as plsc

N = 10000
E = 320000
TWO_N = 2 * N
HID = 128
QCOL = 32
NQ = HID // QCOL
OUT_DIM = 64

NUM_CORES = 2
NUM_TILES = 16
CHUNK = 400
VECS = CHUNK // 16
EDGES_PER_TILE = E // NUM_TILES
CHUNKS_PER_TILE = EDGES_PER_TILE // CHUNK
H_SLICE = 1248
H_TAIL_OFF = H_SLICE * NUM_TILES
H_TAIL = TWO_N - H_TAIL_OFF
B_ROWS = 416


def _sc_pass_body(gather_by_edge, *refs):
    (ni_hbm, ee_hbm, at_hbm, tab0, tab1, tab2, tab3,
     out0, out1, out2, out3,
     bgi, bsi, onesb, hbuf, htail, rows0, rows1, bounce,
     acc_sh, hist_sh, sem0, sem1) = refs

    cid = lax.axis_index("c")
    wid = lax.axis_index("s")
    r0 = wid * H_SLICE
    ebase = wid * EDGES_PER_TILE

    d0 = pltpu.async_copy(ni_hbm.at[pl.ds(ebase, EDGES_PER_TILE)], bgi, sem0)
    d1 = pltpu.async_copy(at_hbm.at[pl.ds(ebase, EDGES_PER_TILE)], bsi, sem1)

    zeros16 = jnp.zeros((16,), jnp.float32)

    def zb_body(i, c):
        r = i // 2
        j = (i % 2) * 16
        bounce[r, pl.ds(j, 16)] = zeros16
        return c

    lax.fori_loop(0, B_ROWS * (QCOL // 16), zb_body, 0)

    def zh_body(i, c):
        hbuf[pl.ds(i * 16, 16)] = zeros16
        return c

    lax.fori_loop(0, H_SLICE // 16, zh_body, 0)
    pltpu.sync_copy(hbuf, hist_sh.at[pl.ds(r0, H_SLICE)])

    @pl.when(wid == 0)
    def _():
        pltpu.sync_copy(hbuf.at[pl.ds(0, H_TAIL)],
                        hist_sh.at[pl.ds(H_TAIL_OFF, H_TAIL)])

    ones16 = jnp.ones((16,), jnp.float32)

    def ob_body(i, c):
        onesb[pl.ds(i * 16, 16)] = ones16
        return c

    lax.fori_loop(0, VECS, ob_body, 0)

    d0.wait()
    d1.wait()

    def ix1_body(i, c):
        s = i * 16
        bgi[pl.ds(s, 16)] = bgi[pl.ds(s, 16)] + bsi[pl.ds(s, 16)] * N
        return c

    lax.fori_loop(0, EDGES_PER_TILE // 16, ix1_body, 0)

    pltpu.async_copy(ee_hbm.at[pl.ds(ebase, EDGES_PER_TILE)], bsi, sem0).wait()

    def ix2_body(i, c):
        s = i * 16
        g = bgi[pl.ds(s, 16)]
        off = jnp.where(g >= N, N, 0).astype(jnp.int32)
        bsi[pl.ds(s, 16)] = bsi[pl.ds(s, 16)] + off
        return c

    lax.fori_loop(0, EDGES_PER_TILE // 16, ix2_body, 0)

    g_all = bsi if gather_by_edge else bgi
    s_all = bgi if gather_by_edge else bsi

    def round_q(tab, out_ref, do_hist):
        def zacc_body(i, c):
            pltpu.sync_copy(bounce, acc_sh.at[pl.ds(r0 + i * B_ROWS, B_ROWS)])
            return c

        lax.fori_loop(0, H_SLICE // B_ROWS, zacc_body, 0)

        @pl.when(wid == 0)
        def _():
            pltpu.sync_copy(bounce.at[pl.ds(0, H_TAIL)],
                            acc_sh.at[pl.ds(H_TAIL_OFF, H_TAIL)])

        plsc.subcore_barrier()

        def fire(c, rows, sem):
            pltpu.async_copy(tab.at[g_all.at[pl.ds(c * CHUNK, CHUNK)]],
                             rows, sem)

        def wait_g(c, rows, sem):
            pltpu.make_async_copy(tab.at[g_all.at[pl.ds(c * CHUNK, CHUNK)]],
                                  rows, sem).wait()

        def scatter(c, rows):
            pltpu.sync_copy(rows, acc_sh.at[s_all.at[pl.ds(c * CHUNK, CHUNK)]],
                            add=True)
            if do_hist:
                pltpu.sync_copy(onesb,
                                hist_sh.at[s_all.at[pl.ds(c * CHUNK, CHUNK)]],
                                add=True)

        fire(0, rows0, sem0)

        def pair_body(p, carry):
            c0 = 2 * p
            c1 = 2 * p + 1
            fire(c1, rows1, sem1)
            wait_g(c0, rows0, sem0)
            scatter(c0, rows0)

            @pl.when(c1 + 1 < CHUNKS_PER_TILE)
            def _():
                fire(c1 + 1, rows0, sem0)

            wait_g(c1, rows1, sem1)
            scatter(c1, rows1)
            return carry

        lax.fori_loop(0, CHUNKS_PER_TILE // 2, pair_body, 0)
        plsc.subcore_barrier()

        if do_hist:
            pltpu.sync_copy(hist_sh.at[pl.ds(r0, H_SLICE)], hbuf)

            @pl.when(wid == 0)
            def _():
                pltpu.sync_copy(hist_sh.at[pl.ds(H_TAIL_OFF, H_TAIL)], htail)

        def wb_body(i, c):
            off = r0 + i * B_ROWS
            pltpu.sync_copy(acc_sh.at[pl.ds(off, B_ROWS)], bounce)

            def sc_body(g, c2):
                base = g * 16
                dvec = hbuf[pl.ds(i * B_ROWS + base, 16)]
                dinv = jnp.where(dvec > 0, 1.0 / dvec, 0.0)
                for l in range(16):
                    s = dinv[l]
                    bounce[base + l, pl.ds(0, 16)] = (
                        bounce[base + l, pl.ds(0, 16)] * s)
                    bounce[base + l, pl.ds(16, 16)] = (
                        bounce[base + l, pl.ds(16, 16)] * s)
                return c2

            lax.fori_loop(0, B_ROWS // 16, sc_body, 0)
            pltpu.sync_copy(bounce, out_ref.at[pl.ds(off, B_ROWS)])
            return c

        lax.fori_loop(0, H_SLICE // B_ROWS, wb_body, 0)

        @pl.when(wid == 0)
        def _():
            pltpu.sync_copy(acc_sh.at[pl.ds(H_TAIL_OFF, H_TAIL)],
                            bounce.at[pl.ds(0, H_TAIL)])

            def sct_body(g, c2):
                base = g * 16
                dvec = htail[pl.ds(base, 16)]
                dinv = jnp.where(dvec > 0, 1.0 / dvec, 0.0)
                for l in range(16):
                    s = dinv[l]
                    bounce[base + l, pl.ds(0, 16)] = (
                        bounce[base + l, pl.ds(0, 16)] * s)
                    bounce[base + l, pl.ds(16, 16)] = (
                        bounce[base + l, pl.ds(16, 16)] * s)
                return c2

            lax.fori_loop(0, H_TAIL // 16, sct_body, 0)
            pltpu.sync_copy(bounce.at[pl.ds(0, H_TAIL)],
                            out_ref.at[pl.ds(H_TAIL_OFF, H_TAIL)])

        lax.fori_loop(0, B_ROWS * (QCOL // 16), zb_body, 0)

    @pl.when(cid == 0)
    def _():
        round_q(tab0, out0, True)
        round_q(tab1, out1, False)

    @pl.when(cid == 1)
    def _():
        round_q(tab2, out2, True)
        round_q(tab3, out3, False)


def _make_sc_pass(gather_by_edge):
    mesh = plsc.VectorSubcoreMesh(core_axis_name="c", subcore_axis_name="s",
                                  num_cores=NUM_CORES, num_subcores=NUM_TILES)
    f32, i32 = jnp.float32, jnp.int32
    out_type = [jax.ShapeDtypeStruct((TWO_N, QCOL), f32) for _ in range(NQ)]
    scratch = [pltpu.VMEM((EDGES_PER_TILE,), i32),
               pltpu.VMEM((EDGES_PER_TILE,), i32),
               pltpu.VMEM((CHUNK,), f32),
               pltpu.VMEM((H_SLICE,), f32),
               pltpu.VMEM((H_TAIL,), f32),
               pltpu.VMEM((CHUNK, QCOL), f32),
               pltpu.VMEM((CHUNK, QCOL), f32),
               pltpu.VMEM((B_ROWS, QCOL), f32),
               pltpu.VMEM_SHARED((TWO_N, QCOL), f32),
               pltpu.VMEM_SHARED((TWO_N,), f32),
               pltpu.SemaphoreType.DMA, pltpu.SemaphoreType.DMA]

    body = functools.partial(_sc_pass_body, gather_by_edge)
    return pl.kernel(body, out_type=out_type, mesh=mesh,
                     scratch_types=scratch,
                     compiler_params=pltpu.CompilerParams(
                         use_tc_tiling_on_sc=False))


_sc_pass1 = _make_sc_pass(gather_by_edge=False)
_sc_pass2 = _make_sc_pass(gather_by_edge=True)


def _xl_body(x_ref, W_ref, o0_ref, o1_ref, o2_ref, o3_ref):
    res = lax.dot_general(x_ref[...], W_ref[0],
                          (((1,), (1,)), ((), ())),
                          preferred_element_type=jnp.float32)
    o0_ref[...] = res[:, 0 * QCOL:1 * QCOL]
    o1_ref[...] = res[:, 1 * QCOL:2 * QCOL]
    o2_ref[...] = res[:, 2 * QCOL:3 * QCOL]
    o3_ref[...] = res[:, 3 * QCOL:4 * QCOL]


def _xl_quarters(x, W01):
    nb = 5
    blk = N // nb
    return pl.pallas_call(
        _xl_body,
        grid=(nb, 2),
        in_specs=[pl.BlockSpec((blk, HID), lambda i, t: (i, 0)),
                  pl.BlockSpec((1, HID, HID), lambda i, t: (t, 0, 0))],
        out_specs=[pl.BlockSpec((blk, QCOL), lambda i, t: (t * 5 + i, 0))
                   for _ in range(NQ)],
        out_shape=[jax.ShapeDtypeStruct((TWO_N, QCOL), jnp.float32)
                   for _ in range(NQ)],
    )(x, W01)


def _dot_t(a, w_ref):
    return lax.dot_general(a, w_ref[...], (((1,), (1,)), ((), ())),
                           preferred_element_type=jnp.float32)


def _final_body(q00, q01, q02, q03, q10, q11, q12, q13, hp_ref,
                Wm, bm, b0r, b1r, Wih, bih, Whh, bhh, Wro, bro,
                hn_ref, pr_ref):
    x0 = jnp.concatenate([q00[...], q01[...], q02[...], q03[...]], axis=1)
    x1 = jnp.concatenate([q10[...], q11[...], q12[...], q13[...]], axis=1)
    Wm0 = Wm[:, :HID]
    Wm1 = Wm[:, HID:]
    bias = (_dot_t(b0r[...], Wm0) + _dot_t(b1r[...], Wm1) + bm[...])
    pre = (lax.dot_general(x0, Wm0, (((1,), (1,)), ((), ())),
                           preferred_element_type=jnp.float32)
           + lax.dot_general(x1, Wm1, (((1,), (1,)), ((), ())),
                             preferred_element_type=jnp.float32)
           + bias)
    h = jnp.maximum(pre, 0.0)
    gi = _dot_t(h, Wih) + bih[...]
    hp = hp_ref[...]
    gh = _dot_t(hp, Whh) + bhh[...]
    r = jax.nn.sigmoid(gi[:, :HID] + gh[:, :HID])
    z = jax.nn.sigmoid(gi[:, HID:2 * HID] + gh[:, HID:2 * HID])
    n = jnp.tanh(gi[:, 2 * HID:] + r * gh[:, 2 * HID:])
    hn = (1.0 - z) * n + z * hp
    hn_ref[...] = hn
    pr_ref[...] = _dot_t(hn, Wro) + bro[...]


def _final(outq, h_prev, W_mix, b_mix, b0, b1,
           W_ih, b_ih, W_hh, b_hh, W_ro, b_ro):
    nb = 10
    blk = N // nb
    full = lambda shape: pl.BlockSpec(shape, lambda i: (0,) * len(shape))
    return pl.pallas_call(
        _final_body,
        grid=(nb,),
        in_specs=[pl.BlockSpec((blk, QCOL), lambda i: (i, 0))
                  for _ in range(NQ)]
        + [pl.BlockSpec((blk, QCOL), lambda i: (nb + i, 0))
           for _ in range(NQ)]
        + [pl.BlockSpec((blk, HID), lambda i: (i, 0)),
           full((HID, 2 * HID)), full((1, HID)), full((1, HID)),
           full((1, HID)),
           full((3 * HID, HID)), full((1, 3 * HID)),
           full((3 * HID, HID)), full((1, 3 * HID)),
           full((OUT_DIM, HID)), full((1, OUT_DIM))],
        out_specs=[pl.BlockSpec((blk, HID), lambda i: (i, 0)),
                   pl.BlockSpec((blk, OUT_DIM), lambda i: (i, 0))],
        out_shape=[jax.ShapeDtypeStruct((N, HID), jnp.float32),
                   jax.ShapeDtypeStruct((N, OUT_DIM), jnp.float32)],
    )(*outq, *outq, h_prev, W_mix, b_mix.reshape(1, HID),
      b0.reshape(1, HID), b1.reshape(1, HID),
      W_ih, b_ih.reshape(1, 3 * HID), W_hh, b_hh.reshape(1, 3 * HID),
      W_ro, b_ro.reshape(1, OUT_DIM))


def kernel(x, edge_index, edge_attr, h_prev, W0, b0, W1, b1, W_mix, b_mix,
           W_ih, W_hh, b_ih, b_hh, W_ro, b_ro):
    node_idx = edge_index[0]
    hedge_idx = edge_index[1]

    xlq = _xl_quarters(x, jnp.stack([W0, W1]))

    efq = _sc_pass1(node_idx, hedge_idx, edge_attr, *xlq)

    outq = _sc_pass2(node_idx, hedge_idx, edge_attr, *efq)

    h_next, pred = _final(outq, h_prev, W_mix, b_mix, b0, b1,
                          W_ih, b_ih, W_hh, b_hh, W_ro, b_ro)
    return (h_next, pred[:, :3])
